# Initial kernel scaffold; baseline (speedup 1.0000x reference)
import jax
import jax.numpy as jnp
from jax.experimental import pallas as pl


def kernel(target, context, neg, u_weight, v_weight):
    raise NotImplementedError("write your pallas kernel here")


# speedup vs baseline: 4.7869x; 4.7869x over previous
import functools

import jax
import jax.numpy as jnp
from jax import lax
from jax.experimental import pallas as pl
from jax.experimental.pallas import tpu as pltpu
from jax.experimental.pallas import tpu_sc as plsc

NC = 2
NS = 16
NW = NC * NS

B = 16384
D = 64
K = 20
J = K + 1
BPW = B // NW
C = 64
NCH = BPW // C
SCOL = 32


def _sc_body(tgt_hbm, cat_hbm, u_hbm, v_hbm, out_hbm,
             tgt_v, idx_v, urows_v, vrows_v, scores_v, sem):
    wid = lax.axis_index("s") * NC + lax.axis_index("c")
    base = wid * BPW
    pltpu.sync_copy(tgt_hbm.at[pl.ds(base, BPW)], tgt_v)
    pltpu.sync_copy(cat_hbm.at[pl.ds(base * J, BPW * J)], idx_v)

    for c in range(NCH):
        cps = [pltpu.async_copy(u_hbm.at[tgt_v.at[pl.ds(c * C, C)]],
                                urows_v, sem)]
        roff = c * C * J
        nfull, tail = (C * J) // 128, (C * J) % 128
        for i in range(nfull):
            cps.append(pltpu.async_copy(
                v_hbm.at[idx_v.at[pl.ds(roff + i * 128, 128)]],
                vrows_v.at[pl.ds(i * 128, 128)], sem))
        if tail:
            cps.append(pltpu.async_copy(
                v_hbm.at[idx_v.at[pl.ds(roff + nfull * 128, tail)]],
                vrows_v.at[pl.ds(nfull * 128, tail)], sem))
        for cp in cps:
            cp.wait()

        lanes = lax.iota(jnp.int32, 16)
        m15 = lanes == 15

        def bbody(b, carry):
            u0 = urows_v[b, pl.ds(0, 16)]
            u1 = urows_v[b, pl.ds(16, 16)]
            u2 = urows_v[b, pl.ds(32, 16)]
            u3 = urows_v[b, pl.ds(48, 16)]
            r0 = b * J
            bfull = jnp.full((16,), 0, jnp.int32) + b
            for j in range(J):
                p = (u0 * vrows_v[r0 + j, pl.ds(0, 16)]
                     + u1 * vrows_v[r0 + j, pl.ds(16, 16)]
                     + u2 * vrows_v[r0 + j, pl.ds(32, 16)]
                     + u3 * vrows_v[r0 + j, pl.ds(48, 16)])
                cs = plsc.cumsum(p)
                plsc.store_scatter(scores_v, [bfull, lanes * 0 + j], cs,
                                   mask=m15)
            return carry

        lax.fori_loop(0, C, bbody, 0)
        pltpu.sync_copy(scores_v, out_hbm.at[pl.ds(base + c * C, C), :])


_sc_scores = functools.partial(
    pl.kernel, _sc_body,
    out_type=jax.ShapeDtypeStruct((B, SCOL), jnp.float32),
    mesh=plsc.VectorSubcoreMesh(core_axis_name="c", subcore_axis_name="s",
                                num_cores=NC, num_subcores=NS),
    compiler_params=pltpu.CompilerParams(needs_layout_passes=False,
                                         use_tc_tiling_on_sc=False),
    scratch_types=[
        pltpu.VMEM((BPW,), jnp.int32),
        pltpu.VMEM((BPW * J,), jnp.int32),
        pltpu.VMEM((C, D), jnp.float32),
        pltpu.VMEM((C * J, D), jnp.float32),
        pltpu.VMEM((C, SCOL), jnp.float32),
        pltpu.SemaphoreType.DMA,
    ],
)()


def _tc_finish_body(s_ref, o_ref):
    x = s_ref[...]
    col = lax.broadcasted_iota(jnp.int32, x.shape, 1)
    y = jnp.where(col == 0, -x, x)
    sp = jnp.maximum(y, 0.0) + jnp.log1p(jnp.exp(-jnp.abs(y)))
    sp = jnp.where(col < J, sp, 0.0)
    o_ref[...] = jnp.sum(sp, axis=1)


_TCR = 2048


def _tc_finish(scores):
    return pl.pallas_call(
        _tc_finish_body,
        grid=(B // _TCR,),
        in_specs=[pl.BlockSpec((_TCR, SCOL), lambda i: (i, 0))],
        out_specs=pl.BlockSpec((_TCR,), lambda i: (i,)),
        out_shape=jax.ShapeDtypeStruct((B,), jnp.float32),
    )(scores)


def kernel(target, context, neg, u_weight, v_weight):
    tgt = target.astype(jnp.int32)
    cat = jnp.concatenate(
        [context.astype(jnp.int32)[:, None], neg.astype(jnp.int32)],
        axis=1).reshape(-1)
    scores = _sc_scores(tgt, cat, u_weight, v_weight)
    return _tc_finish(scores)
